# baseline (speedup 1.0000x reference)
import jax
import jax.numpy as jnp
from jax import lax
from jax.experimental import pallas as pl
from jax.experimental.pallas import tpu as pltpu
from jax.experimental.pallas import tpu_sc as plsc

NC = 2
NS = 16
NW = NC * NS
HT = 25
BT = 32
TILES_PER_TEC = HT * BT // NW


def _body(xt_hbm, tp_hbm, out_hbm, xidx_v, pidx_v, poff_v, rows0, rows1,
          ot0, ot1, gsem0, gsem1, osem0, osem1):
    wid = lax.axis_index("s") * NC + lax.axis_index("c")
    base_t = wid * TILES_PER_TEC

    lane = lax.iota(jnp.int32, 16)
    bidx = [lane + 16 * g for g in range(8)]

    rows = (rows0, rows1)
    ots = (ot0, ot1)
    gsems = (gsem0, gsem1)
    osems = (osem0, osem1)

    @pl.loop(0, TILES_PER_TEC)
    def _tile(k):
        t = base_t + k
        ht = lax.shift_right_logical(t, 5)
        bt = lax.bitwise_and(t, BT - 1)

        pltpu.sync_copy(xt_hbm.at[pl.ds(ht * 8, 8), pl.ds(bt * 128, 128)],
                        xidx_v)
        for r8 in range(8):
            for g8 in range(8):
                v = xidx_v[r8, pl.ds(g8 * 16, 16)]
                pidx_v[r8, pl.ds(g8 * 16, 16)] = lax.shift_right_logical(v, 1)
                poff_v[r8, pl.ds(g8 * 16, 16)] = lax.shift_left(
                    lax.bitwise_and(v, 1), 6)

        def assemble(r_dyn, buf):
            for g in range(8):
                pv = poff_v[r_dyn, pl.ds(g * 16, 16)]
                for dd in range(64):
                    ots[buf][dd, pl.ds(g * 16, 16)] = (
                        plsc.load_gather(rows[buf], [bidx[g], pv + dd]))

        pltpu.async_copy(tp_hbm.at[pidx_v.at[0]], rows[0], gsem0)

        @pl.loop(0, 8, step=2)
        def _r(r):
            for sub in range(2):
                rr = r + sub
                buf = sub
                nbuf = 1 - sub
                nxt = lax.rem(rr + 1, 8)
                pltpu.async_copy(tp_hbm.at[pidx_v.at[nxt]], rows[nbuf],
                                 gsems[nbuf])
                pltpu.make_async_copy(
                    tp_hbm.at[pidx_v.at[0]], rows[buf], gsems[buf]).wait()

                @pl.when(jnp.logical_or(k > 0, r > 0))
                def _wait_prev():
                    pltpu.make_async_copy(
                        ots[buf], out_hbm.at[0, :, pl.ds(0, 128)],
                        osems[buf]).wait()

                assemble(rr, buf)
                pltpu.async_copy(
                    ots[buf],
                    out_hbm.at[ht * 8 + rr, :, pl.ds(bt * 128, 128)],
                    osems[buf])

        pltpu.make_async_copy(
            tp_hbm.at[pidx_v.at[0]], rows[0], gsem0).wait()

    pltpu.make_async_copy(
        out_hbm.at[0, :, pl.ds(0, 128)], ot0, osem0).wait()
    pltpu.make_async_copy(
        out_hbm.at[0, :, pl.ds(0, 128)], ot1, osem1).wait()


def kernel(x, table):
    mesh = plsc.VectorSubcoreMesh(core_axis_name="c", subcore_axis_name="s")
    out = pl.kernel(
        _body,
        out_type=jax.ShapeDtypeStruct((200, 64, 4096), jnp.float32),
        mesh=mesh,
        scratch_types=[
            pltpu.VMEM((8, 128), jnp.int32),
            pltpu.VMEM((8, 128), jnp.int32),
            pltpu.VMEM((8, 128), jnp.int32),
            pltpu.VMEM((128, 128), jnp.float32),
            pltpu.VMEM((128, 128), jnp.float32),
            pltpu.VMEM((64, 128), jnp.float32),
            pltpu.VMEM((64, 128), jnp.float32),
            pltpu.SemaphoreType.DMA,
            pltpu.SemaphoreType.DMA,
            pltpu.SemaphoreType.DMA,
            pltpu.SemaphoreType.DMA,
        ],
        compiler_params=pltpu.CompilerParams(
            use_tc_tiling_on_sc=True, needs_layout_passes=False),
    )(x.T, table.reshape(500000, 128))
    return out.transpose(2, 0, 1)
